# baseline (speedup 1.0000x reference)
import functools

import jax
import jax.numpy as jnp
from jax import lax
from jax.experimental import pallas as pl
from jax.experimental.pallas import tpu as pltpu
from jax.experimental.pallas import tpu_sc as plsc

B, M, N, D = 8, 4096, 1024, 128
K = 3
ML = K * N
ZROW = B * ML

NC, NS = 2, 16
NW = NC * NS
TPW = (B * N) // NW
CHUNK = 64
NCHUNK = TPW // CHUNK
LANES = 16


def _matmul_relu_body(a_ref, w_ref, o_ref):
    i = pl.program_id(0)

    @pl.when(i < B)
    def _():
        o_ref[...] = jnp.maximum(
            lax.dot_general(a_ref[0], w_ref[...], (((1,), (1,)), ((), ())),
                            preferred_element_type=jnp.float32),
            0.0,
        )

    @pl.when(i == B)
    def _():
        o_ref[...] = jnp.zeros_like(o_ref)


def _matmul_relu(feats, w):
    return pl.pallas_call(
        _matmul_relu_body,
        grid=(B + 1,),
        in_specs=[
            pl.BlockSpec((1, ML, D), lambda i: (jnp.minimum(i, B - 1), 0, 0)),
            pl.BlockSpec((D, D), lambda i: (0, 0)),
        ],
        out_specs=pl.BlockSpec((ML, D), lambda i: (i, 0)),
        out_shape=jax.ShapeDtypeStruct(((B + 1) * ML, D), jnp.float32),
    )(feats, w)


def _idx_w_body(lens_ref, idx_ref, w_ref):
    lens = lens_ref[...]
    lensf = lens.astype(jnp.float32)
    row = lax.broadcasted_iota(jnp.int32, (N, N), 0)
    col = lax.broadcasted_iota(jnp.int32, (N, N), 1)
    tri = (row <= col).astype(jnp.float32)
    csum = lax.dot_general(lensf, tri, (((1,), (0,)), ((), ())),
                           preferred_element_type=jnp.float32)
    start = csum - lensf
    gbase = lax.broadcasted_iota(jnp.int32, (B, N), 0).astype(jnp.float32)
    gbase = gbase * float(ML)
    w_ref[...] = 1.0 / jnp.maximum(lensf, 1.0)
    for k in range(K):
        live = start + gbase + float(k)
        idx_ref[k] = jnp.where(lens > k, live, float(ZROW)).astype(jnp.int32)


def _idx_w(lens):
    return pl.pallas_call(
        _idx_w_body,
        out_shape=(
            jax.ShapeDtypeStruct((K, B, N), jnp.int32),
            jax.ShapeDtypeStruct((B, N), jnp.float32),
        ),
    )(lens)


_SC_MESH = plsc.VectorSubcoreMesh(
    core_axis_name="c", subcore_axis_name="s", num_cores=NC, num_subcores=NS,
)


@functools.partial(
    pl.kernel,
    out_type=jax.ShapeDtypeStruct((B * N, D), jnp.float32),
    mesh=_SC_MESH,
    compiler_params=pltpu.CompilerParams(needs_layout_passes=False),
    scratch_types=[
        pltpu.VMEM((NCHUNK, CHUNK), jnp.int32),
        pltpu.VMEM((NCHUNK, CHUNK), jnp.int32),
        pltpu.VMEM((NCHUNK, CHUNK), jnp.int32),
        pltpu.VMEM((TPW,), jnp.float32),
        pltpu.VMEM((2, CHUNK, D), jnp.float32),
        pltpu.VMEM((2, CHUNK, D), jnp.float32),
        pltpu.VMEM((2, CHUNK, D), jnp.float32),
        pltpu.VMEM((2, CHUNK, D), jnp.float32),
        pltpu.SemaphoreType.DMA,
        pltpu.SemaphoreType.DMA,
        pltpu.SemaphoreType.DMA,
        pltpu.SemaphoreType.DMA,
    ],
)
def _pool_sc(x_hbm, i0_hbm, i1_hbm, i2_hbm, w_hbm, out_hbm,
             i0, i1, i2, w, r0, r1, r2, ov, gsa, gsb, osa, osb):
    wid = lax.axis_index("s") * NC + lax.axis_index("c")
    base = wid * TPW
    pltpu.sync_copy(w_hbm.at[pl.ds(base, TPW)], w)
    for h in range(NCHUNK):
        sl = pl.ds(base + h * CHUNK, CHUNK)
        pltpu.sync_copy(i0_hbm.at[sl], i0.at[h])
        pltpu.sync_copy(i1_hbm.at[sl], i1.at[h])
        pltpu.sync_copy(i2_hbm.at[sl], i2.at[h])
    gsems = (gsa, gsb)
    osems = (osa, osb)

    def issue_gathers(h):
        bb = h % 2
        return (
            pltpu.async_copy(x_hbm.at[i0.at[h]], r0.at[bb], gsems[bb]),
            pltpu.async_copy(x_hbm.at[i1.at[h]], r1.at[bb], gsems[bb]),
            pltpu.async_copy(x_hbm.at[i2.at[h]], r2.at[bb], gsems[bb]),
        )

    pending_g = {0: issue_gathers(0)}
    pending_o = {}
    for h in range(NCHUNK):
        bb = h % 2
        if h + 1 < NCHUNK:
            pending_g[h + 1] = issue_gathers(h + 1)
        for c in pending_g.pop(h):
            c.wait()
        if h >= 2:
            pending_o.pop(h - 2).wait()

        def tok_body(t, carry, _h=h, _bb=bb):
            ts = jnp.full((LANES,), t + _h * CHUNK, jnp.int32)
            wsv = plsc.load_gather(w, [ts])
            for j in range(D // LANES):
                dsl = pl.ds(j * LANES, LANES)
                ov[_bb, t, dsl] = (r0[_bb, t, dsl] + r1[_bb, t, dsl]
                                   + r2[_bb, t, dsl]) * wsv
            return carry

        lax.fori_loop(0, CHUNK, tok_body, 0)

        pending_o[h] = pltpu.async_copy(
            ov.at[bb], out_hbm.at[pl.ds(base + h * CHUNK, CHUNK)], osems[bb])
    for h in sorted(pending_o):
        pending_o[h].wait()


def kernel(atom_feats, atom_mask, molecule_atom_lens, W):
    del atom_mask
    b, m, d = atom_feats.shape
    n = molecule_atom_lens.shape[1]
    assert (b, m, n, d) == (B, M, N, D)
    lens = molecule_atom_lens.astype(jnp.int32)
    x = _matmul_relu(atom_feats, W)
    idx, w = _idx_w(lens)
    idxf = idx.reshape(K, b * n)
    out = _pool_sc(x, idxf[0], idxf[1], idxf[2], w.reshape(b * n))
    return out.reshape(b, n, d)


# speedup vs baseline: 8.6968x; 8.6968x over previous
import functools

import jax
import jax.numpy as jnp
from jax import lax
from jax.experimental import pallas as pl
from jax.experimental.pallas import tpu as pltpu
from jax.experimental.pallas import tpu_sc as plsc

B, M, N, D = 8, 4096, 1024, 128
K = 3
ML = K * N
ZROW = B * ML

NC, NS = 2, 16
NW = NC * NS
TPW = (B * N) // NW
CHUNK = 64
NCHUNK = TPW // CHUNK
LANES = 16


def _matmul_relu_body(a_ref, w_ref, o_ref):
    i = pl.program_id(0)

    @pl.when(i < B)
    def _():
        o_ref[...] = jnp.maximum(
            lax.dot_general(a_ref[0], w_ref[...], (((1,), (1,)), ((), ())),
                            preferred_element_type=jnp.float32),
            0.0,
        )

    @pl.when(i == B)
    def _():
        o_ref[...] = jnp.zeros_like(o_ref)


def _matmul_relu(feats, w):
    return pl.pallas_call(
        _matmul_relu_body,
        grid=(B + 1,),
        in_specs=[
            pl.BlockSpec((1, ML, D), lambda i: (jnp.minimum(i, B - 1), 0, 0)),
            pl.BlockSpec((D, D), lambda i: (0, 0)),
        ],
        out_specs=pl.BlockSpec((ML, D), lambda i: (i, 0)),
        out_shape=jax.ShapeDtypeStruct(((B + 1) * ML, D), jnp.float32),
    )(feats, w)


def _idx_w_body(lens_ref, idx_ref, w_ref):
    lens = lens_ref[...]
    lensf = lens.astype(jnp.float32)
    row = lax.broadcasted_iota(jnp.int32, (N, N), 0)
    col = lax.broadcasted_iota(jnp.int32, (N, N), 1)
    tri = (row <= col).astype(jnp.float32)
    csum = lax.dot_general(lensf, tri, (((1,), (0,)), ((), ())),
                           preferred_element_type=jnp.float32)
    start = csum - lensf
    gbase = lax.broadcasted_iota(jnp.int32, (B, N), 0).astype(jnp.float32)
    gbase = gbase * float(ML)
    w_ref[...] = 1.0 / jnp.maximum(lensf, 1.0)
    bi = lax.broadcasted_iota(jnp.int32, (B, N), 0)
    ti = lax.broadcasted_iota(jnp.int32, (B, N), 1)
    for k in range(K):
        live = (start + gbase + float(k)).astype(jnp.int32)
        spread = ti * K + k + bi * (ML // B)
        spread = jnp.where(spread >= ML, spread - ML, spread)
        idx_ref[k] = jnp.where(lens > k, live, ZROW + spread)


def _idx_w(lens):
    return pl.pallas_call(
        _idx_w_body,
        out_shape=(
            jax.ShapeDtypeStruct((K, B, N), jnp.int32),
            jax.ShapeDtypeStruct((B, N), jnp.float32),
        ),
    )(lens)


_SC_MESH = plsc.VectorSubcoreMesh(
    core_axis_name="c", subcore_axis_name="s", num_cores=NC, num_subcores=NS,
)


@functools.partial(
    pl.kernel,
    out_type=jax.ShapeDtypeStruct((B * N, D), jnp.float32),
    mesh=_SC_MESH,
    compiler_params=pltpu.CompilerParams(needs_layout_passes=False),
    scratch_types=[
        pltpu.VMEM((NCHUNK, CHUNK), jnp.int32),
        pltpu.VMEM((NCHUNK, CHUNK), jnp.int32),
        pltpu.VMEM((NCHUNK, CHUNK), jnp.int32),
        pltpu.VMEM((TPW,), jnp.float32),
        pltpu.VMEM((2, CHUNK, D), jnp.float32),
        pltpu.VMEM((2, CHUNK, D), jnp.float32),
        pltpu.VMEM((2, CHUNK, D), jnp.float32),
        pltpu.VMEM((2, CHUNK, D), jnp.float32),
        pltpu.SemaphoreType.DMA,
        pltpu.SemaphoreType.DMA,
        pltpu.SemaphoreType.DMA,
        pltpu.SemaphoreType.DMA,
    ],
)
def _pool_sc(x_hbm, i0_hbm, i1_hbm, i2_hbm, w_hbm, out_hbm,
             i0, i1, i2, w, r0, r1, r2, ov, gsa, gsb, osa, osb):
    wid = lax.axis_index("s") * NC + lax.axis_index("c")
    base = wid * TPW
    pltpu.sync_copy(w_hbm.at[pl.ds(base, TPW)], w)
    for h in range(NCHUNK):
        sl = pl.ds(base + h * CHUNK, CHUNK)
        pltpu.sync_copy(i0_hbm.at[sl], i0.at[h])
        pltpu.sync_copy(i1_hbm.at[sl], i1.at[h])
        pltpu.sync_copy(i2_hbm.at[sl], i2.at[h])
    gsems = (gsa, gsb)
    osems = (osa, osb)

    def issue_gathers(h):
        bb = h % 2
        return (
            pltpu.async_copy(x_hbm.at[i0.at[h]], r0.at[bb], gsems[bb]),
            pltpu.async_copy(x_hbm.at[i1.at[h]], r1.at[bb], gsems[bb]),
            pltpu.async_copy(x_hbm.at[i2.at[h]], r2.at[bb], gsems[bb]),
        )

    pending_g = {0: issue_gathers(0)}
    pending_o = {}
    for h in range(NCHUNK):
        bb = h % 2
        if h + 1 < NCHUNK:
            pending_g[h + 1] = issue_gathers(h + 1)
        for c in pending_g.pop(h):
            c.wait()
        if h >= 2:
            pending_o.pop(h - 2).wait()

        def tok_body(t, carry, _h=h, _bb=bb):
            ts = jnp.full((LANES,), t + _h * CHUNK, jnp.int32)
            wsv = plsc.load_gather(w, [ts])
            for j in range(D // LANES):
                dsl = pl.ds(j * LANES, LANES)
                ov[_bb, t, dsl] = (r0[_bb, t, dsl] + r1[_bb, t, dsl]
                                   + r2[_bb, t, dsl]) * wsv
            return carry

        lax.fori_loop(0, CHUNK, tok_body, 0)

        pending_o[h] = pltpu.async_copy(
            ov.at[bb], out_hbm.at[pl.ds(base + h * CHUNK, CHUNK)], osems[bb])
    for h in sorted(pending_o):
        pending_o[h].wait()


def kernel(atom_feats, atom_mask, molecule_atom_lens, W):
    del atom_mask
    b, m, d = atom_feats.shape
    n = molecule_atom_lens.shape[1]
    assert (b, m, n, d) == (B, M, N, D)
    lens = molecule_atom_lens.astype(jnp.int32)
    x = _matmul_relu(atom_feats, W)
    idx, w = _idx_w(lens)
    idxf = idx.reshape(K, b * n)
    out = _pool_sc(x, idxf[0], idxf[1], idxf[2], w.reshape(b * n))
    return out.reshape(b, n, d)


# speedup vs baseline: 10.8318x; 1.2455x over previous
import functools

import jax
import jax.numpy as jnp
from jax import lax
from jax.experimental import pallas as pl
from jax.experimental.pallas import tpu as pltpu
from jax.experimental.pallas import tpu_sc as plsc

B, M, N, D = 8, 4096, 1024, 128
K = 3
ML = K * N
ZROW = B * ML

NC, NS = 2, 16
NW = NC * NS
TPW = (B * N) // NW
CHUNK = 32
NCHUNK = TPW // CHUNK
GROWS = K * CHUNK
LANES = 16


def _fused_tc_body(a_ref, w_ref, lens_ref, o_ref, idx_ref, wout_ref):
    i = pl.program_id(0)

    @pl.when(i < B)
    def _():
        o_ref[...] = jnp.maximum(
            lax.dot_general(a_ref[0], w_ref[...], (((1,), (1,)), ((), ())),
                            preferred_element_type=jnp.float32),
            0.0,
        )

    @pl.when(i == B)
    def _():
        o_ref[...] = jnp.zeros_like(o_ref)
        lens = lens_ref[...]
        lensf = lens.astype(jnp.float32)
        row = lax.broadcasted_iota(jnp.int32, (N, N), 0)
        col = lax.broadcasted_iota(jnp.int32, (N, N), 1)
        tri = (row <= col).astype(jnp.float32)
        csum = lax.dot_general(lensf, tri, (((1,), (0,)), ((), ())),
                               preferred_element_type=jnp.float32)
        start = csum - lensf
        gbase = lax.broadcasted_iota(jnp.int32, (B, N), 0).astype(jnp.float32)
        gbase = gbase * float(ML)
        wout_ref[...] = (1.0 / jnp.maximum(lensf, 1.0)).reshape(NW, TPW)
        bi = lax.broadcasted_iota(jnp.int32, (B, N), 0)
        ti = lax.broadcasted_iota(jnp.int32, (B, N), 1)
        idx_k = []
        for k in range(K):
            live = (start + gbase + float(k)).astype(jnp.int32)
            spread = ti * K + k + bi * (ML // B)
            spread = jnp.where(spread >= ML, spread - ML, spread)
            idx_k.append(jnp.where(lens > k, live, ZROW + spread)
                         .reshape(NW, TPW))
        parts = []
        for h in range(NCHUNK):
            for k in range(K):
                parts.append(
                    lax.slice(idx_k[k], (0, h * CHUNK), (NW, (h + 1) * CHUNK)))
        idx_ref[...] = jnp.concatenate(parts, axis=1)


def _fused_tc(feats, w, lens):
    return pl.pallas_call(
        _fused_tc_body,
        grid=(B + 1,),
        in_specs=[
            pl.BlockSpec((1, ML, D), lambda i: (jnp.minimum(i, B - 1), 0, 0)),
            pl.BlockSpec((D, D), lambda i: (0, 0)),
            pl.BlockSpec((B, N), lambda i: (0, 0)),
        ],
        out_specs=[
            pl.BlockSpec((ML, D), lambda i: (i, 0)),
            pl.BlockSpec((NW, NCHUNK * GROWS), lambda i: (0, 0)),
            pl.BlockSpec((NW, TPW), lambda i: (0, 0)),
        ],
        out_shape=[
            jax.ShapeDtypeStruct(((B + 1) * ML, D), jnp.float32),
            jax.ShapeDtypeStruct((NW, NCHUNK * GROWS), jnp.int32),
            jax.ShapeDtypeStruct((NW, TPW), jnp.float32),
        ],
    )(feats, w, lens)


_SC_MESH = plsc.VectorSubcoreMesh(
    core_axis_name="c", subcore_axis_name="s", num_cores=NC, num_subcores=NS,
)


@functools.partial(
    pl.kernel,
    out_type=jax.ShapeDtypeStruct((B * N, D), jnp.float32),
    mesh=_SC_MESH,
    compiler_params=pltpu.CompilerParams(needs_layout_passes=False),
    scratch_types=[
        pltpu.VMEM((NCHUNK, GROWS), jnp.int32),
        pltpu.VMEM((TPW,), jnp.float32),
        pltpu.VMEM((2, GROWS, D), jnp.float32),
        pltpu.VMEM((2, CHUNK, D), jnp.float32),
        pltpu.SemaphoreType.DMA,
        pltpu.SemaphoreType.DMA,
        pltpu.SemaphoreType.DMA,
        pltpu.SemaphoreType.DMA,
    ],
)
def _pool_sc(x_hbm, idx_hbm, w_hbm, out_hbm, iv, w, r, ov, gsa, gsb, osa, osb):
    wid = lax.axis_index("s") * NC + lax.axis_index("c")
    base = wid * TPW
    pltpu.sync_copy(idx_hbm.at[wid], iv)
    pltpu.sync_copy(w_hbm.at[wid], w)
    gsems = (gsa, gsb)
    osems = (osa, osb)

    def issue_gather(h):
        bb = h % 2
        return pltpu.async_copy(x_hbm.at[iv.at[h]], r.at[bb], gsems[bb])

    pending_g = {0: issue_gather(0)}
    pending_o = {}
    for h in range(NCHUNK):
        bb = h % 2
        if h + 1 < NCHUNK:
            pending_g[h + 1] = issue_gather(h + 1)
        pending_g.pop(h).wait()
        if h >= 2:
            pending_o.pop(h - 2).wait()

        def tok_body(t, carry, _h=h, _bb=bb):
            ts = jnp.full((LANES,), t + _h * CHUNK, jnp.int32)
            wsv = plsc.load_gather(w, [ts])
            for j in range(D // LANES):
                dsl = pl.ds(j * LANES, LANES)
                ov[_bb, t, dsl] = (r[_bb, t, dsl]
                                   + r[_bb, CHUNK + t, dsl]
                                   + r[_bb, 2 * CHUNK + t, dsl]) * wsv
            return carry

        lax.fori_loop(0, CHUNK, tok_body, 0)
        pending_o[h] = pltpu.async_copy(
            ov.at[bb], out_hbm.at[pl.ds(base + h * CHUNK, CHUNK)], osems[bb])
    for h in sorted(pending_o):
        pending_o[h].wait()


def kernel(atom_feats, atom_mask, molecule_atom_lens, W):
    del atom_mask
    b, m, d = atom_feats.shape
    n = molecule_atom_lens.shape[1]
    assert (b, m, n, d) == (B, M, N, D)
    lens = molecule_atom_lens.astype(jnp.int32)
    x, idx, w = _fused_tc(atom_feats, W, lens)
    out = _pool_sc(x, idx.reshape(NW, NCHUNK, GROWS), w)
    return out.reshape(b, n, d)


# speedup vs baseline: 11.3664x; 1.0494x over previous
import functools

import jax
import jax.numpy as jnp
from jax import lax
from jax.experimental import pallas as pl
from jax.experimental.pallas import tpu as pltpu
from jax.experimental.pallas import tpu_sc as plsc

B, M, N, D = 8, 4096, 1024, 128
K = 3
ML = K * N
ZROW = B * ML

NC, NS = 2, 16
NW = NC * NS
TPW = (B * N) // NW
CHUNK = 32
NCHUNK = TPW // CHUNK
GROWS = K * CHUNK
LANES = 16


def _fused_tc_body(a_ref, w_ref, lens_ref, o_ref, idx_ref, wout_ref):
    i = pl.program_id(0)

    @pl.when(i < B)
    def _():
        o_ref[...] = jnp.maximum(
            lax.dot_general(a_ref[0], w_ref[...], (((1,), (1,)), ((), ())),
                            preferred_element_type=jnp.float32),
            0.0,
        )

    @pl.when(i == B)
    def _():
        o_ref[...] = jnp.zeros_like(o_ref)
        lens = lens_ref[...]
        lensf = lens.astype(jnp.float32)
        row = lax.broadcasted_iota(jnp.int32, (N, N), 0)
        col = lax.broadcasted_iota(jnp.int32, (N, N), 1)
        tri = (row <= col).astype(jnp.float32)
        csum = lax.dot_general(lensf, tri, (((1,), (0,)), ((), ())),
                               preferred_element_type=jnp.float32)
        start = csum - lensf
        gbase = lax.broadcasted_iota(jnp.int32, (B, N), 0).astype(jnp.float32)
        gbase = gbase * float(ML)
        wout_ref[...] = (1.0 / jnp.maximum(lensf, 1.0)).reshape(NW, TPW)
        bi = lax.broadcasted_iota(jnp.int32, (B, N), 0)
        ti = lax.broadcasted_iota(jnp.int32, (B, N), 1)
        idx_k = []
        for k in range(K):
            live = (start + gbase + float(k)).astype(jnp.int32)
            spread = ti * K + k + bi * (ML // B)
            spread = jnp.where(spread >= ML, spread - ML, spread)
            idx_k.append(jnp.where(lens > k, live, ZROW + spread)
                         .reshape(NW, TPW))
        parts = []
        for h in range(NCHUNK):
            for k in range(K):
                parts.append(
                    lax.slice(idx_k[k], (0, h * CHUNK), (NW, (h + 1) * CHUNK)))
        idx_ref[...] = jnp.concatenate(parts, axis=1)


def _fused_tc(feats, w, lens):
    return pl.pallas_call(
        _fused_tc_body,
        grid=(B + 1,),
        in_specs=[
            pl.BlockSpec((1, ML, D), lambda i: (jnp.minimum(i, B - 1), 0, 0)),
            pl.BlockSpec((D, D), lambda i: (0, 0)),
            pl.BlockSpec((B, N), lambda i: (0, 0)),
        ],
        out_specs=[
            pl.BlockSpec((ML, D), lambda i: (i, 0)),
            pl.BlockSpec((NW, NCHUNK * GROWS), lambda i: (0, 0)),
            pl.BlockSpec((NW, TPW), lambda i: (0, 0)),
        ],
        out_shape=[
            jax.ShapeDtypeStruct(((B + 1) * ML, D), jnp.float32),
            jax.ShapeDtypeStruct((NW, NCHUNK * GROWS), jnp.int32),
            jax.ShapeDtypeStruct((NW, TPW), jnp.float32),
        ],
    )(feats, w, lens)


_SC_MESH = plsc.VectorSubcoreMesh(
    core_axis_name="c", subcore_axis_name="s", num_cores=NC, num_subcores=NS,
)


@functools.partial(
    pl.kernel,
    out_type=jax.ShapeDtypeStruct((B * N, D), jnp.float32),
    mesh=_SC_MESH,
    compiler_params=pltpu.CompilerParams(needs_layout_passes=False),
    scratch_types=[
        pltpu.VMEM((NCHUNK, GROWS), jnp.int32),
        pltpu.VMEM((TPW,), jnp.float32),
        pltpu.VMEM((3, GROWS, D), jnp.float32),
        pltpu.VMEM((2, CHUNK, D), jnp.float32),
        pltpu.SemaphoreType.DMA,
        pltpu.SemaphoreType.DMA,
        pltpu.SemaphoreType.DMA,
        pltpu.SemaphoreType.DMA,
        pltpu.SemaphoreType.DMA,
    ],
)
def _pool_sc(x_hbm, idx_hbm, w_hbm, out_hbm, iv, w, r, ov,
             gsa, gsb, gsc, osa, osb):
    wid = lax.axis_index("s") * NC + lax.axis_index("c")
    base = wid * TPW
    pltpu.sync_copy(idx_hbm.at[wid], iv)
    pltpu.sync_copy(w_hbm.at[wid], w)
    gsems = (gsa, gsb, gsc)
    osems = (osa, osb)

    def issue_gather(h):
        bb = h % 3
        return pltpu.async_copy(x_hbm.at[iv.at[h]], r.at[bb], gsems[bb])

    pending_g = {0: issue_gather(0), 1: issue_gather(1)}
    pending_o = {}
    for h in range(NCHUNK):
        bb = h % 3
        ob = h % 2
        if h + 2 < NCHUNK:
            pending_g[h + 2] = issue_gather(h + 2)
        pending_g.pop(h).wait()
        if h >= 2:
            pending_o.pop(h - 2).wait()

        def tok_body(i, carry, _h=h, _bb=bb, _ob=ob):
            for u in range(2):
                t = i * 2 + u
                ts = jnp.full((LANES,), t + _h * CHUNK, jnp.int32)
                wsv = plsc.load_gather(w, [ts])
                for j in range(D // LANES):
                    dsl = pl.ds(j * LANES, LANES)
                    ov[_ob, t, dsl] = (r[_bb, t, dsl]
                                       + r[_bb, CHUNK + t, dsl]
                                       + r[_bb, 2 * CHUNK + t, dsl]) * wsv
            return carry

        lax.fori_loop(0, CHUNK // 2, tok_body, 0)
        pending_o[h] = pltpu.async_copy(
            ov.at[ob], out_hbm.at[pl.ds(base + h * CHUNK, CHUNK)], osems[ob])
    for h in sorted(pending_o):
        pending_o[h].wait()


def kernel(atom_feats, atom_mask, molecule_atom_lens, W):
    del atom_mask
    b, m, d = atom_feats.shape
    n = molecule_atom_lens.shape[1]
    assert (b, m, n, d) == (B, M, N, D)
    lens = molecule_atom_lens.astype(jnp.int32)
    x, idx, w = _fused_tc(atom_feats, W, lens)
    out = _pool_sc(x, idx.reshape(NW, NCHUNK, GROWS), w)
    return out.reshape(b, n, d)


# speedup vs baseline: 11.8538x; 1.0429x over previous
import functools

import jax
import jax.numpy as jnp
from jax import lax
from jax.experimental import pallas as pl
from jax.experimental.pallas import tpu as pltpu
from jax.experimental.pallas import tpu_sc as plsc

B, M, N, D = 8, 4096, 1024, 128
K = 3
ML = K * N
ZROW = B * ML

NC, NS = 2, 16
NW = NC * NS
TPW = (B * N) // NW
CHUNK = 32
NCHUNK = TPW // CHUNK
GROWS = K * CHUNK
LANES = 16


def _fused_tc_body(a_ref, w_ref, lens_ref, o_ref, idx_ref, wout_ref):
    i = pl.program_id(0)

    @pl.when(i < B)
    def _():
        o_ref[...] = jnp.maximum(
            lax.dot_general(a_ref[0], w_ref[...], (((1,), (1,)), ((), ())),
                            preferred_element_type=jnp.float32),
            0.0,
        )

    @pl.when(i == B)
    def _():
        o_ref[...] = jnp.zeros_like(o_ref)
        lens = lens_ref[...]
        lensf = lens.astype(jnp.float32)
        row = lax.broadcasted_iota(jnp.int32, (N, N), 0)
        col = lax.broadcasted_iota(jnp.int32, (N, N), 1)
        tri = (row <= col).astype(jnp.float32)
        csum = lax.dot_general(lensf, tri, (((1,), (0,)), ((), ())),
                               preferred_element_type=jnp.float32)
        start = csum - lensf
        gbase = lax.broadcasted_iota(jnp.int32, (B, N), 0).astype(jnp.float32)
        gbase = gbase * float(ML)
        wout_ref[...] = (1.0 / jnp.maximum(lensf, 1.0)).reshape(NW, TPW)
        bi = lax.broadcasted_iota(jnp.int32, (B, N), 0)
        ti = lax.broadcasted_iota(jnp.int32, (B, N), 1)
        idx_k = []
        for k in range(K):
            live = (start + gbase + float(k)).astype(jnp.int32)
            spread = ti * K + k + bi * (ML // B)
            spread = jnp.where(spread >= ML, spread - ML, spread)
            idx_k.append(jnp.where(lens > k, live, ZROW + spread)
                         .reshape(NW, TPW))
        parts = []
        for h in range(NCHUNK):
            for k in range(K):
                parts.append(
                    lax.slice(idx_k[k], (0, h * CHUNK), (NW, (h + 1) * CHUNK)))
        idx_ref[...] = jnp.concatenate(parts, axis=1)


def _fused_tc(feats, w, lens):
    return pl.pallas_call(
        _fused_tc_body,
        grid=(B + 1,),
        in_specs=[
            pl.BlockSpec((1, ML, D), lambda i: (jnp.minimum(i, B - 1), 0, 0)),
            pl.BlockSpec((D, D), lambda i: (0, 0)),
            pl.BlockSpec((B, N), lambda i: (0, 0)),
        ],
        out_specs=[
            pl.BlockSpec((ML, D), lambda i: (i, 0)),
            pl.BlockSpec((NW, NCHUNK * GROWS), lambda i: (0, 0)),
            pl.BlockSpec((NW, TPW), lambda i: (0, 0)),
        ],
        out_shape=[
            jax.ShapeDtypeStruct(((B + 1) * ML, D), jnp.float32),
            jax.ShapeDtypeStruct((NW, NCHUNK * GROWS), jnp.int32),
            jax.ShapeDtypeStruct((NW, TPW), jnp.float32),
        ],
    )(feats, w, lens)


_SC_MESH = plsc.VectorSubcoreMesh(
    core_axis_name="c", subcore_axis_name="s", num_cores=NC, num_subcores=NS,
)


@functools.partial(
    pl.kernel,
    out_type=jax.ShapeDtypeStruct((B * N, D), jnp.float32),
    mesh=_SC_MESH,
    compiler_params=pltpu.CompilerParams(needs_layout_passes=False),
    scratch_types=[
        pltpu.VMEM((NCHUNK, GROWS), jnp.int32),
        pltpu.VMEM((TPW,), jnp.float32),
        pltpu.VMEM((4, GROWS, D), jnp.float32),
        pltpu.VMEM((2, CHUNK, D), jnp.float32),
        pltpu.SemaphoreType.DMA,
        pltpu.SemaphoreType.DMA,
        pltpu.SemaphoreType.DMA,
        pltpu.SemaphoreType.DMA,
        pltpu.SemaphoreType.DMA,
        pltpu.SemaphoreType.DMA,
    ],
)
def _pool_sc(x_hbm, idx_hbm, w_hbm, out_hbm, iv, w, r, ov,
             gsa, gsb, gsc, gsd, osa, osb):
    wid = lax.axis_index("s") * NC + lax.axis_index("c")
    base = wid * TPW
    pltpu.sync_copy(idx_hbm.at[wid], iv)
    pltpu.sync_copy(w_hbm.at[wid], w)
    gsems = (gsa, gsb, gsc, gsd)
    osems = (osa, osb)

    def issue_gather(h):
        bb = h % 4
        return pltpu.async_copy(x_hbm.at[iv.at[h]], r.at[bb], gsems[bb])

    pending_g = {0: issue_gather(0), 1: issue_gather(1), 2: issue_gather(2)}
    pending_o = {}
    for h in range(NCHUNK):
        bb = h % 4
        ob = h % 2
        if h + 3 < NCHUNK:
            pending_g[h + 3] = issue_gather(h + 3)
        pending_g.pop(h).wait()
        if h >= 2:
            pending_o.pop(h - 2).wait()

        def tok_body(i, carry, _h=h, _bb=bb, _ob=ob):
            for u in range(2):
                t = i * 2 + u
                ts = jnp.full((LANES,), t + _h * CHUNK, jnp.int32)
                wsv = plsc.load_gather(w, [ts])
                for j in range(D // LANES):
                    dsl = pl.ds(j * LANES, LANES)
                    ov[_ob, t, dsl] = (r[_bb, t, dsl]
                                       + r[_bb, CHUNK + t, dsl]
                                       + r[_bb, 2 * CHUNK + t, dsl]) * wsv
            return carry

        lax.fori_loop(0, CHUNK // 2, tok_body, 0)
        pending_o[h] = pltpu.async_copy(
            ov.at[ob], out_hbm.at[pl.ds(base + h * CHUNK, CHUNK)], osems[ob])
    for h in sorted(pending_o):
        pending_o[h].wait()


def kernel(atom_feats, atom_mask, molecule_atom_lens, W):
    del atom_mask
    b, m, d = atom_feats.shape
    n = molecule_atom_lens.shape[1]
    assert (b, m, n, d) == (B, M, N, D)
    lens = molecule_atom_lens.astype(jnp.int32)
    x, idx, w = _fused_tc(atom_feats, W, lens)
    out = _pool_sc(x, idx.reshape(NW, NCHUNK, GROWS), w)
    return out.reshape(b, n, d)


# speedup vs baseline: 12.2476x; 1.0332x over previous
import functools

import jax
import jax.numpy as jnp
from jax import lax
from jax.experimental import pallas as pl
from jax.experimental.pallas import tpu as pltpu
from jax.experimental.pallas import tpu_sc as plsc

B, M, N, D = 8, 4096, 1024, 128
K = 3
ML = K * N
ZROW = B * ML

NC, NS = 2, 16
NW = NC * NS
TPW = (B * N) // NW
CHUNK = 32
NCHUNK = TPW // CHUNK
GROWS = K * CHUNK
LANES = 16


def _fused_tc_body(a_ref, w_ref, lens_ref, o_ref, idx_ref, wout_ref):
    i = pl.program_id(0)

    @pl.when(i < B)
    def _():
        o_ref[...] = jnp.maximum(
            lax.dot_general(a_ref[0], w_ref[...], (((1,), (1,)), ((), ())),
                            preferred_element_type=jnp.float32),
            0.0,
        )

    @pl.when(i == B)
    def _():
        o_ref[...] = jnp.zeros_like(o_ref)
        lens = lens_ref[...]
        lensf = lens.astype(jnp.float32)
        row = lax.broadcasted_iota(jnp.int32, (N, N), 0)
        col = lax.broadcasted_iota(jnp.int32, (N, N), 1)
        tri = (row <= col).astype(jnp.float32)
        csum = lax.dot_general(lensf, tri, (((1,), (0,)), ((), ())),
                               preferred_element_type=jnp.float32)
        start = csum - lensf
        gbase = lax.broadcasted_iota(jnp.int32, (B, N), 0).astype(jnp.float32)
        gbase = gbase * float(ML)
        wout_ref[...] = (1.0 / jnp.maximum(lensf, 1.0)).reshape(NW, TPW)
        bi = lax.broadcasted_iota(jnp.int32, (B, N), 0)
        ti = lax.broadcasted_iota(jnp.int32, (B, N), 1)
        idx_k = []
        for k in range(K):
            live = (start + gbase + float(k)).astype(jnp.int32)
            spread = ti * K + k + bi * (ML // B)
            spread = jnp.where(spread >= ML, spread - ML, spread)
            idx_k.append(jnp.where(lens > k, live, ZROW + spread)
                         .reshape(NW, TPW))
        parts = []
        for h in range(NCHUNK):
            for k in range(K):
                parts.append(
                    lax.slice(idx_k[k], (0, h * CHUNK), (NW, (h + 1) * CHUNK)))
        idx_ref[...] = jnp.concatenate(parts, axis=1)


def _fused_tc(feats, w, lens):
    return pl.pallas_call(
        _fused_tc_body,
        grid=(B + 1,),
        in_specs=[
            pl.BlockSpec((1, ML, D), lambda i: (jnp.minimum(i, B - 1), 0, 0)),
            pl.BlockSpec((D, D), lambda i: (0, 0)),
            pl.BlockSpec((B, N), lambda i: (0, 0)),
        ],
        out_specs=[
            pl.BlockSpec((ML, D), lambda i: (i, 0)),
            pl.BlockSpec((NW, NCHUNK * GROWS), lambda i: (0, 0)),
            pl.BlockSpec((NW, TPW), lambda i: (0, 0)),
        ],
        out_shape=[
            jax.ShapeDtypeStruct(((B + 1) * ML, D), jnp.float32),
            jax.ShapeDtypeStruct((NW, NCHUNK * GROWS), jnp.int32),
            jax.ShapeDtypeStruct((NW, TPW), jnp.float32),
        ],
    )(feats, w, lens)


_SC_MESH = plsc.VectorSubcoreMesh(
    core_axis_name="c", subcore_axis_name="s", num_cores=NC, num_subcores=NS,
)


@functools.partial(
    pl.kernel,
    out_type=jax.ShapeDtypeStruct((B * N, D), jnp.float32),
    mesh=_SC_MESH,
    compiler_params=pltpu.CompilerParams(needs_layout_passes=False),
    scratch_types=[
        pltpu.VMEM((NCHUNK, GROWS), jnp.int32),
        pltpu.VMEM((TPW,), jnp.float32),
        pltpu.VMEM((NCHUNK, GROWS, D), jnp.float32),
        pltpu.VMEM((2, CHUNK, D), jnp.float32),
        [pltpu.SemaphoreType.DMA] * NCHUNK,
        pltpu.SemaphoreType.DMA,
        pltpu.SemaphoreType.DMA,
    ],
)
def _pool_sc(x_hbm, idx_hbm, w_hbm, out_hbm, iv, w, r, ov,
             gsems, osa, osb):
    wid = lax.axis_index("s") * NC + lax.axis_index("c")
    base = wid * TPW
    pltpu.sync_copy(idx_hbm.at[wid], iv)
    pltpu.sync_copy(w_hbm.at[wid], w)
    osems = (osa, osb)

    def issue_gather(h):
        return pltpu.async_copy(x_hbm.at[iv.at[h]], r.at[h], gsems[h])

    pending_g = {h: issue_gather(h) for h in range(NCHUNK)}
    pending_o = {}
    for h in range(NCHUNK):
        bb = h
        ob = h % 2
        pending_g.pop(h).wait()
        if h >= 2:
            pending_o.pop(h - 2).wait()

        def tok_body(i, carry, _h=h, _bb=bb, _ob=ob):
            for u in range(4):
                t = i * 4 + u
                ts = jnp.full((LANES,), t + _h * CHUNK, jnp.int32)
                wsv = plsc.load_gather(w, [ts])
                for j in range(D // LANES):
                    dsl = pl.ds(j * LANES, LANES)
                    ov[_ob, t, dsl] = (r[_bb, t, dsl]
                                       + r[_bb, CHUNK + t, dsl]
                                       + r[_bb, 2 * CHUNK + t, dsl]) * wsv
            return carry

        lax.fori_loop(0, CHUNK // 4, tok_body, 0)
        pending_o[h] = pltpu.async_copy(
            ov.at[ob], out_hbm.at[pl.ds(base + h * CHUNK, CHUNK)], osems[ob])
    for h in sorted(pending_o):
        pending_o[h].wait()


def kernel(atom_feats, atom_mask, molecule_atom_lens, W):
    del atom_mask
    b, m, d = atom_feats.shape
    n = molecule_atom_lens.shape[1]
    assert (b, m, n, d) == (B, M, N, D)
    lens = molecule_atom_lens.astype(jnp.int32)
    x, idx, w = _fused_tc(atom_feats, W, lens)
    out = _pool_sc(x, idx.reshape(NW, NCHUNK, GROWS), w)
    return out.reshape(b, n, d)
